# baseline (speedup 1.0000x reference)
import functools

import jax
import jax.numpy as jnp
from jax import lax
from jax.experimental import pallas as pl
from jax.experimental.pallas import tpu as pltpu
from jax.experimental.pallas import tpu_sc as plsc

DIM_HALF = 64
DC = 2 * DIM_HALF
EXT = 9216
SEQ = 8192
ROW_BLK = 128
NUM_BLKS = EXT // ROW_BLK

NC = 2
NS = 16
NW = NC * NS
BPW = SEQ // NW


def _cache_body(invf_ref, out_ref, tab_ref):
    i = pl.program_id(0)

    @pl.when(i == 0)
    def _build_tables():
        l = (lax.broadcasted_iota(jnp.int32, (ROW_BLK, DIM_HALF), 0)
             .astype(jnp.float32))
        ang_lo = l * invf_ref[...]
        ang_hi = ang_lo * float(ROW_BLK)
        tab_ref[0:ROW_BLK, :] = jnp.concatenate(
            [jnp.cos(ang_lo), jnp.sin(ang_lo)], axis=1)
        tab_ref[ROW_BLK:2 * ROW_BLK, :] = jnp.concatenate(
            [jnp.cos(ang_hi), jnp.sin(ang_hi)], axis=1)

    hi = tab_ref[pl.ds(ROW_BLK + i, 1), :]
    ch = hi[:, :DIM_HALF]
    sh = hi[:, DIM_HALF:]
    cl = tab_ref[0:ROW_BLK, :DIM_HALF]
    sl = tab_ref[0:ROW_BLK, DIM_HALF:]
    out_ref[...] = jnp.concatenate(
        [ch * cl - sh * sl, sh * cl + ch * sl], axis=1)


def _build_cache(inv_freq):
    invf2d = inv_freq.reshape(1, DIM_HALF)
    return pl.pallas_call(
        _cache_body,
        grid=(NUM_BLKS,),
        in_specs=[pl.BlockSpec((1, DIM_HALF), lambda i: (0, 0))],
        out_specs=pl.BlockSpec((ROW_BLK, DC), lambda i: (i, 0)),
        out_shape=jax.ShapeDtypeStruct((EXT, DC), jnp.float32),
        scratch_shapes=[pltpu.VMEM((2 * ROW_BLK, DC), jnp.float32)],
    )(invf2d)


@functools.cache
def _make_sc_gather():
    mesh = plsc.VectorSubcoreMesh(core_axis_name="c", subcore_axis_name="s")

    @functools.partial(
        pl.kernel,
        mesh=mesh,
        out_type=jax.ShapeDtypeStruct((SEQ, DC), jnp.float32),
        scratch_types=[
            pltpu.VMEM((BPW,), jnp.int32),
            pltpu.VMEM((BPW, DC), jnp.float32),
            pltpu.SemaphoreType.DMA,
        ],
    )
    def _sc_gather(cache_hbm, pos_hbm, out_hbm, idx_v, rows_v, sem):
        wid = lax.axis_index("s") * NC + lax.axis_index("c")
        base = wid * BPW
        pltpu.sync_copy(pos_hbm.at[pl.ds(base, BPW)], idx_v)
        pltpu.async_copy(cache_hbm.at[idx_v], rows_v, sem).wait()
        pltpu.sync_copy(rows_v, out_hbm.at[pl.ds(base, BPW)])

    return _sc_gather


def kernel(positions, inv_freq):
    cache = _build_cache(inv_freq)
    pos32 = positions.astype(jnp.int32)
    both = _make_sc_gather()(cache, pos32)
    return (both[:, :DIM_HALF], both[:, DIM_HALF:])


# speedup vs baseline: 1.4847x; 1.4847x over previous
import functools

import jax
import jax.numpy as jnp
from jax import lax
from jax.experimental import pallas as pl
from jax.experimental.pallas import tpu as pltpu
from jax.experimental.pallas import tpu_sc as plsc

DIM_HALF = 64
DC = 2 * DIM_HALF
EXT = 9216
SEQ = 8192
ROW_BLK = 128
NUM_BLKS = EXT // ROW_BLK

NC = 2
NS = 16
NW = NC * NS
BPW = SEQ // NW


def _tables_body(invf_ref, tab_ref):
    l = (lax.broadcasted_iota(jnp.int32, (ROW_BLK, DIM_HALF), 0)
         .astype(jnp.float32))
    ang_lo = l * invf_ref[...]
    ang_hi = ang_lo * float(ROW_BLK)
    tab_ref[0:ROW_BLK, :] = jnp.concatenate(
        [jnp.cos(ang_lo), jnp.sin(ang_lo)], axis=1)
    tab_ref[ROW_BLK:2 * ROW_BLK, :] = jnp.concatenate(
        [jnp.cos(ang_hi), jnp.sin(ang_hi)], axis=1)


HPB = 8
CACHE_BLK = HPB * ROW_BLK
N_CACHE_BLKS = EXT // CACHE_BLK


def _combine_body(tab_ref, out_ref):
    i = pl.program_id(0)
    hi = tab_ref[pl.ds(ROW_BLK + HPB * i, HPB), :]
    ch = hi[:, :DIM_HALF].reshape(HPB, 1, DIM_HALF)
    sh = hi[:, DIM_HALF:].reshape(HPB, 1, DIM_HALF)
    lo = tab_ref[0:ROW_BLK, :]
    cl = lo[:, :DIM_HALF].reshape(1, ROW_BLK, DIM_HALF)
    sl = lo[:, DIM_HALF:].reshape(1, ROW_BLK, DIM_HALF)
    cos_c = ch * cl - sh * sl
    sin_c = sh * cl + ch * sl
    out = jnp.concatenate([cos_c, sin_c], axis=2)
    out_ref[...] = out.reshape(CACHE_BLK, DC)


def _build_cache(inv_freq):
    invf2d = inv_freq.reshape(1, DIM_HALF)
    tab = pl.pallas_call(
        _tables_body,
        out_shape=jax.ShapeDtypeStruct((2 * ROW_BLK, DC), jnp.float32),
    )(invf2d)
    return pl.pallas_call(
        _combine_body,
        grid=(N_CACHE_BLKS,),
        in_specs=[pl.BlockSpec((2 * ROW_BLK, DC), lambda i: (0, 0))],
        out_specs=pl.BlockSpec((CACHE_BLK, DC), lambda i: (i, 0)),
        out_shape=jax.ShapeDtypeStruct((EXT, DC), jnp.float32),
    )(tab)


@functools.cache
def _make_sc_gather():
    mesh = plsc.VectorSubcoreMesh(core_axis_name="c", subcore_axis_name="s")

    @functools.partial(
        pl.kernel,
        mesh=mesh,
        out_type=jax.ShapeDtypeStruct((SEQ, DC), jnp.float32),
        scratch_types=[
            pltpu.VMEM((BPW,), jnp.int32),
            pltpu.VMEM((BPW, DC), jnp.float32),
            pltpu.SemaphoreType.DMA,
        ],
    )
    def _sc_gather(cache_hbm, pos_hbm, out_hbm, idx_v, rows_v, sem):
        wid = lax.axis_index("s") * NC + lax.axis_index("c")
        base = wid * BPW
        pltpu.sync_copy(pos_hbm.at[pl.ds(base, BPW)], idx_v)
        pltpu.async_copy(cache_hbm.at[idx_v], rows_v, sem).wait()
        pltpu.sync_copy(rows_v, out_hbm.at[pl.ds(base, BPW)])

    return _sc_gather


def kernel(positions, inv_freq):
    cache = _build_cache(inv_freq)
    pos32 = positions.astype(jnp.int32)
    both = _make_sc_gather()(cache, pos32)
    return (both[:, :DIM_HALF], both[:, DIM_HALF:])
